# Initial kernel scaffold; baseline (speedup 1.0000x reference)
import jax
import jax.numpy as jnp
from jax.experimental import pallas as pl


def kernel(f, k, idx, conv1_w, conv1_b, conv2_w, conv2_b, d0_w, d0_b, d1_w, d1_b):
    raise NotImplementedError("write your pallas kernel here")


# speedup vs baseline: 8.5788x; 8.5788x over previous
import functools

import jax
import jax.numpy as jnp
from jax import lax
from jax.experimental import pallas as pl
from jax.experimental.pallas import tpu as pltpu
from jax.experimental.pallas import tpu_sc as plsc

N = 10000
K = 32
C = 128
E = N * K


_TN1 = 1000


def _k1_body(f_ref, c1w_ref, d0w_ref, d1w_ref, d0b_ref, d1b_ref,
             g_ref, wf_ref, bf_ref):
    f_blk = f_ref[...]
    g_ref[...] = lax.dot_general(
        f_blk, c1w_ref[...], (((1,), (1,)), ((), ())),
        preferred_element_type=jnp.float32)
    d0w = d0w_ref[...]
    d1w = d1w_ref[...]
    w0 = d0w[:, :C] + d0w[:, C:]
    d1a = d1w[:, :C]
    wf_ref[...] = lax.dot_general(
        d1a, w0, (((1,), (0,)), ((), ())),
        preferred_element_type=jnp.float32) + d1w[:, C:]
    bf_ref[...] = lax.dot_general(
        d1a, d0b_ref[...], (((1,), (0,)), ((), ())),
        preferred_element_type=jnp.float32) + d1b_ref[...]


def _stage1(f2, conv1_w, d0_w, d1_w, d0_b_col, d1_b_col):
    return pl.pallas_call(
        _k1_body,
        grid=(N // _TN1,),
        in_specs=[
            pl.BlockSpec((_TN1, C), lambda i: (i, 0)),
            pl.BlockSpec((C, C), lambda i: (0, 0)),
            pl.BlockSpec((C, 2 * C), lambda i: (0, 0)),
            pl.BlockSpec((C, 2 * C), lambda i: (0, 0)),
            pl.BlockSpec((C, 1), lambda i: (0, 0)),
            pl.BlockSpec((C, 1), lambda i: (0, 0)),
        ],
        out_specs=[
            pl.BlockSpec((_TN1, C), lambda i: (i, 0)),
            pl.BlockSpec((C, C), lambda i: (0, 0)),
            pl.BlockSpec((C, 1), lambda i: (0, 0)),
        ],
        out_shape=[
            jax.ShapeDtypeStruct((N, C), jnp.float32),
            jax.ShapeDtypeStruct((C, C), jnp.float32),
            jax.ShapeDtypeStruct((C, 1), jnp.float32),
        ],
    )(f2, conv1_w, d0_w, d1_w, d0_b_col, d1_b_col)


_NC = 2
_NS = 16
_NW = _NC * _NS
_PW = E // _NW
_CH = 128
_NFULL = _PW // _CH
_TAIL = _PW - _NFULL * _CH


def _sc_gather_body(g_hbm, idx_hbm, out_hbm, idx_v, rows_v, tail_v,
                    insem0, insem1, outsem0, outsem1):
    wid = lax.axis_index("s") * _NC + lax.axis_index("c")
    wbase = pl.multiple_of(wid * _PW, _PW)
    insem = (insem0, insem1)
    outsem = (outsem0, outsem1)

    pltpu.sync_copy(idx_hbm.at[pl.ds(wbase, _PW)], idx_v)

    def gather_descr(t, slot):
        off = pl.multiple_of(t * _CH, _CH)
        return pltpu.make_async_copy(
            g_hbm.at[idx_v.at[pl.ds(off, _CH)]], rows_v.at[slot], insem[slot])

    def wb_descr(t, slot):
        off = pl.multiple_of(wbase + t * _CH, _CH)
        return pltpu.make_async_copy(
            rows_v.at[slot], out_hbm.at[pl.ds(off, _CH)], outsem[slot])

    def start(t, slot):
        @pl.when(t < _NFULL)
        def _():
            gather_descr(t, slot).start()

    def drain(t, slot):
        gather_descr(t, slot).wait()
        wb_descr(t, slot).start()

    def wait_out(t, slot):
        wb_descr(t, slot).wait()

    start(0, 0)

    def body(tt, _):
        for b in range(2):
            t = 2 * tt + b
            nslot = 1 - b
            @pl.when(t >= 1)
            def _():
                wait_out(t - 1, nslot)
            start(t + 1, nslot)
            drain(t, b)
        return 0

    lax.fori_loop(0, _NFULL // 2, body, 0)
    wait_out(_NFULL - 1, (_NFULL - 1) % 2)

    pltpu.make_async_copy(
        g_hbm.at[idx_v.at[pl.ds(_NFULL * _CH, _TAIL)]], tail_v,
        insem0).start()
    pltpu.make_async_copy(
        g_hbm.at[idx_v.at[pl.ds(_NFULL * _CH, _TAIL)]], tail_v,
        insem0).wait()
    pltpu.sync_copy(tail_v, out_hbm.at[pl.ds(wbase + _NFULL * _CH, _TAIL)])


def _stage2(g, idx_flat):
    mesh = plsc.VectorSubcoreMesh(core_axis_name="c", subcore_axis_name="s")
    run = functools.partial(
        pl.kernel,
        mesh=mesh,
        out_type=jax.ShapeDtypeStruct((E, C), jnp.float32),
        scratch_types=[
            pltpu.VMEM((_PW,), jnp.int32),
            pltpu.VMEM((2, _CH, C), jnp.float32),
            pltpu.VMEM((_TAIL, C), jnp.float32),
            pltpu.SemaphoreType.DMA,
            pltpu.SemaphoreType.DMA,
            pltpu.SemaphoreType.DMA,
            pltpu.SemaphoreType.DMA,
        ],
    )(_sc_gather_body)
    return run(g, idx_flat)


_TN3 = 200
_TE3 = _TN3 * K


def _k3_body(gath_ref, f_ref, c1w_ref, c2w_ref, bsum_ref, wf_ref, bf_ref,
             out_ref):
    f_blk = f_ref[...]
    w12 = c2w_ref[...] - c1w_ref[...]
    base = lax.dot_general(
        f_blk, w12, (((1,), (1,)), ((), ())),
        preferred_element_type=jnp.float32) + bsum_ref[...]
    g3 = gath_ref[...].reshape(_TN3, K, C)
    y = jnp.maximum(g3 + base[:, None, :], 0.0).reshape(_TE3, C)
    zt = lax.dot_general(
        wf_ref[...], y, (((1,), (1,)), ((), ())),
        preferred_element_type=jnp.float32)
    out_ref[...] = zt + bf_ref[...]


def _stage3(gathered, f2, conv1_w, conv2_w, bsum, wf, bf_col):
    return pl.pallas_call(
        _k3_body,
        grid=(N // _TN3,),
        in_specs=[
            pl.BlockSpec((_TE3, C), lambda i: (i, 0)),
            pl.BlockSpec((_TN3, C), lambda i: (i, 0)),
            pl.BlockSpec((C, C), lambda i: (0, 0)),
            pl.BlockSpec((C, C), lambda i: (0, 0)),
            pl.BlockSpec((1, C), lambda i: (0, 0)),
            pl.BlockSpec((C, C), lambda i: (0, 0)),
            pl.BlockSpec((C, 1), lambda i: (0, 0)),
        ],
        out_specs=pl.BlockSpec((C, _TE3), lambda i: (0, i)),
        out_shape=jax.ShapeDtypeStruct((C, E), jnp.float32),
    )(gathered, f2, conv1_w, conv2_w, bsum, wf, bf_col)


def kernel(f, k, idx, conv1_w, conv1_b, conv2_w, conv2_b,
           d0_w, d0_b, d1_w, d1_b):
    f2 = f.reshape(N, C)
    idx_flat = idx.reshape(E).astype(jnp.int32)
    bsum = (conv1_b + conv2_b).reshape(1, C)

    g, wf, bf_col = _stage1(f2, conv1_w, d0_w, d1_w,
                            d0_b.reshape(C, 1), d1_b.reshape(C, 1))
    gathered = _stage2(g, idx_flat)
    out2d = _stage3(gathered, f2, conv1_w, conv2_w, bsum, wf, bf_col)
    return out2d.reshape(1, C, N, K), idx
